# baseline (speedup 1.0000x reference)
import jax
import jax.numpy as jnp
from jax import lax
from jax.experimental import pallas as pl
from jax.experimental.pallas import tpu as pltpu
from jax.experimental.pallas import tpu_sc as plsc

_B = 16384
_D = 2048
_C = 10
_B_SC = 4096
_B_TC = _B - _B_SC
_N_TILES = 32
_ROWS_PER_TILE = _B_SC // _N_TILES
_CHUNK = 16
_NCH = _ROWS_PER_TILE // _CHUNK
_LANES = 16


def _tc_kernel(f_ref, w_ref, b_ref, o_ref):
    acc = lax.dot_general(
        f_ref[...].astype(jnp.bfloat16), w_ref[...].astype(jnp.bfloat16),
        dimension_numbers=(((1,), (1,)), ((), ())),
        preferred_element_type=jnp.float32,
    )
    o_ref[...] = acc + b_ref[...]


def _tc_part(feat, W, b2):
    Bm = 1024
    return pl.pallas_call(
        _tc_kernel,
        grid=(_B_TC // Bm,),
        in_specs=[
            pl.BlockSpec((Bm, _D), lambda i: (i, 0)),
            pl.BlockSpec((_C, _D), lambda i: (0, 0)),
            pl.BlockSpec((1, _C), lambda i: (0, 0)),
        ],
        out_specs=pl.BlockSpec((Bm, _C), lambda i: (i, 0)),
        out_shape=jax.ShapeDtypeStruct((_B_TC, _C), jnp.float32),
        compiler_params=pltpu.CompilerParams(
            dimension_semantics=("arbitrary",),
        ),
    )(feat, W, b2)


def _sc_body(feat_hbm, w_hbm, b_hbm, out_hbm, wv, bv,
             fch0, fch1, ov0, ov1, fsem, osem):
    wid = lax.axis_index("s") * 2 + lax.axis_index("c")
    base = wid * _ROWS_PER_TILE

    pltpu.sync_copy(w_hbm, wv)
    pltpu.sync_copy(b_hbm, bv)
    bias = bv[...]
    lanes = lax.iota(jnp.int32, _LANES)
    zero = jnp.zeros((_LANES,), jnp.float32)
    fbufs = (fch0, fch1)
    obufs = (ov0, ov1)

    def in_copy(ch, buf, sem):
        row0 = base + ch * _CHUNK
        return pltpu.make_async_copy(
            feat_hbm.at[pl.ds(_B_TC + row0, _CHUNK), :], buf, sem)

    def out_copy(ch, buf, sem):
        row0 = base + ch * _CHUNK
        return pltpu.make_async_copy(
            buf, out_hbm.at[pl.ds(row0, _CHUNK), :], sem)

    def compute_chunk(fch, ov):
        for g in range(_CHUNK // 4):
            rows = (4 * g, 4 * g + 1, 4 * g + 2, 4 * g + 3)

            def d_body(d, accs, _rows=rows):
                sl = pl.ds(d * _LANES, _LANES)
                fs = [fch[r, sl] for r in _rows]
                new = list(accs)
                for c in range(_C):
                    wc = wv[c, sl]
                    for ri in range(4):
                        k = ri * _C + c
                        new[k] = new[k] + fs[ri] * wc
                return tuple(new)

            init = tuple(jnp.zeros((_LANES,), jnp.float32)
                         for _ in range(4 * _C))
            accs = lax.fori_loop(0, _D // _LANES, d_body, init, unroll=2)

            for ri in range(4):
                res = bias
                for c in range(_C):
                    s = jnp.sum(accs[ri * _C + c])
                    res = res + jnp.where(lanes == c,
                                          jnp.full((_LANES,), s), zero)
                ov[rows[ri]] = res

    in_copy(0, fch0, fsem.at[0]).start()

    def ch_body(ch, parity):
        fb = fbufs
        ob = obufs

        def run(par):
            @pl.when(ch + 1 < _NCH)
            def _ahead():
                in_copy(ch + 1, fb[1 - par], fsem.at[1 - par]).start()

            in_copy(ch, fb[par], fsem.at[par]).wait()

            @pl.when(ch >= 2)
            def _drain_out():
                out_copy(ch - 2, ob[par], osem.at[par]).wait()

            compute_chunk(fb[par], ob[par])
            out_copy(ch, ob[par], osem.at[par]).start()

        @pl.when(parity == 0)
        def _p0():
            run(0)

        @pl.when(parity == 1)
        def _p1():
            run(1)

        return 1 - parity

    lax.fori_loop(0, _NCH, ch_body, 0)

    out_copy(_NCH - 2, obufs[(_NCH - 2) % 2], osem.at[(_NCH - 2) % 2]).wait()
    out_copy(_NCH - 1, obufs[(_NCH - 1) % 2], osem.at[(_NCH - 1) % 2]).wait()


_sc_part = pl.kernel(
    _sc_body,
    out_type=jax.ShapeDtypeStruct((_B_SC, _LANES), jnp.float32),
    mesh=plsc.VectorSubcoreMesh(core_axis_name="c", subcore_axis_name="s"),
    scratch_types=[
        pltpu.VMEM((_C, _D), jnp.float32),
        pltpu.VMEM((_LANES,), jnp.float32),
        pltpu.VMEM((_CHUNK, _D), jnp.float32),
        pltpu.VMEM((_CHUNK, _D), jnp.float32),
        pltpu.VMEM((_CHUNK, _LANES), jnp.float32),
        pltpu.VMEM((_CHUNK, _LANES), jnp.float32),
        pltpu.SemaphoreType.DMA((2,)),
        pltpu.SemaphoreType.DMA((2,)),
    ],
    compiler_params=pltpu.CompilerParams(needs_layout_passes=False),
)


def kernel(feat, W, b):
    b2 = b.reshape(1, _C)
    b16 = jnp.pad(b, (0, _LANES - _C))
    tc_out = _tc_part(feat, W, b2)
    sc_out = _sc_part(feat, W, b16)
    return jnp.concatenate([tc_out, sc_out[:, :_C]], axis=0)


# speedup vs baseline: 2.3649x; 2.3649x over previous
import jax
import jax.numpy as jnp
from jax import lax
from jax.experimental import pallas as pl
from jax.experimental.pallas import tpu as pltpu
from jax.experimental.pallas import tpu_sc as plsc

_B = 16384
_D = 2048
_C = 10
_B_SC = 2048
_B_TC = _B - _B_SC
_N_TILES = 32
_ROWS_PER_TILE = _B_SC // _N_TILES
_CHUNK = 16
_NCH = _ROWS_PER_TILE // _CHUNK
_LANES = 16


def _tc_kernel(f_ref, w_ref, b_ref, o_ref):
    acc = lax.dot_general(
        f_ref[...].astype(jnp.bfloat16), w_ref[...].astype(jnp.bfloat16),
        dimension_numbers=(((1,), (1,)), ((), ())),
        preferred_element_type=jnp.float32,
    )
    o_ref[...] = acc + b_ref[...]


def _tc_part(feat, W, b2):
    Bm = 1024
    return pl.pallas_call(
        _tc_kernel,
        grid=(_B_TC // Bm,),
        in_specs=[
            pl.BlockSpec((Bm, _D), lambda i: (i, 0)),
            pl.BlockSpec((_C, _D), lambda i: (0, 0)),
            pl.BlockSpec((1, _C), lambda i: (0, 0)),
        ],
        out_specs=pl.BlockSpec((Bm, _C), lambda i: (i, 0)),
        out_shape=jax.ShapeDtypeStruct((_B_TC, _C), jnp.float32),
        compiler_params=pltpu.CompilerParams(
            dimension_semantics=("arbitrary",),
        ),
    )(feat, W, b2)


def _sc_body(feat_hbm, w_hbm, b_hbm, out_hbm, wv, bv,
             fch0, fch1, ov0, ov1, fsem, osem):
    wid = lax.axis_index("s") * 2 + lax.axis_index("c")
    base = wid * _ROWS_PER_TILE

    pltpu.sync_copy(w_hbm, wv)
    pltpu.sync_copy(b_hbm, bv)
    bias = bv[...]
    lanes = lax.iota(jnp.int32, _LANES)
    zero = jnp.zeros((_LANES,), jnp.float32)
    fbufs = (fch0, fch1)
    obufs = (ov0, ov1)

    def in_copy(ch, buf, sem):
        row0 = base + ch * _CHUNK
        return pltpu.make_async_copy(
            feat_hbm.at[pl.ds(_B_TC + row0, _CHUNK), :], buf, sem)

    def out_copy(ch, buf, sem):
        row0 = base + ch * _CHUNK
        return pltpu.make_async_copy(
            buf, out_hbm.at[pl.ds(row0, _CHUNK), :], sem)

    def compute_chunk(fch, ov):
        for g in range(_CHUNK // 2):
            rows = (2 * g, 2 * g + 1)

            def d_body(d, accs, _rows=rows):
                sl = pl.ds(d * _LANES, _LANES)
                fs = [fch[r, sl] for r in _rows]
                new = list(accs)
                for c in range(_C):
                    wc = wv[c, sl]
                    for ri in range(2):
                        k = ri * _C + c
                        new[k] = new[k] + fs[ri] * wc
                return tuple(new)

            init = tuple(jnp.zeros((_LANES,), jnp.float32)
                         for _ in range(2 * _C))
            accs = lax.fori_loop(0, _D // _LANES, d_body, init, unroll=2)

            for ri in range(2):
                res = bias
                for c in range(_C):
                    s = jnp.sum(accs[ri * _C + c])
                    res = res + jnp.where(lanes == c,
                                          jnp.full((_LANES,), s), zero)
                ov[rows[ri]] = res

    in_copy(0, fch0, fsem.at[0]).start()

    def ch_body(ch, parity):
        fb = fbufs
        ob = obufs

        def run(par):
            @pl.when(ch + 1 < _NCH)
            def _ahead():
                in_copy(ch + 1, fb[1 - par], fsem.at[1 - par]).start()

            in_copy(ch, fb[par], fsem.at[par]).wait()

            @pl.when(ch >= 2)
            def _drain_out():
                out_copy(ch - 2, ob[par], osem.at[par]).wait()

            compute_chunk(fb[par], ob[par])
            out_copy(ch, ob[par], osem.at[par]).start()

        @pl.when(parity == 0)
        def _p0():
            run(0)

        @pl.when(parity == 1)
        def _p1():
            run(1)

        return 1 - parity

    lax.fori_loop(0, _NCH, ch_body, 0)

    out_copy(_NCH - 2, obufs[(_NCH - 2) % 2], osem.at[(_NCH - 2) % 2]).wait()
    out_copy(_NCH - 1, obufs[(_NCH - 1) % 2], osem.at[(_NCH - 1) % 2]).wait()


_sc_part = pl.kernel(
    _sc_body,
    out_type=jax.ShapeDtypeStruct((_B_SC, _LANES), jnp.float32),
    mesh=plsc.VectorSubcoreMesh(core_axis_name="c", subcore_axis_name="s"),
    scratch_types=[
        pltpu.VMEM((_C, _D), jnp.float32),
        pltpu.VMEM((_LANES,), jnp.float32),
        pltpu.VMEM((_CHUNK, _D), jnp.float32),
        pltpu.VMEM((_CHUNK, _D), jnp.float32),
        pltpu.VMEM((_CHUNK, _LANES), jnp.float32),
        pltpu.VMEM((_CHUNK, _LANES), jnp.float32),
        pltpu.SemaphoreType.DMA((2,)),
        pltpu.SemaphoreType.DMA((2,)),
    ],
    compiler_params=pltpu.CompilerParams(needs_layout_passes=False),
)


def kernel(feat, W, b):
    b2 = b.reshape(1, _C)
    b16 = jnp.pad(b, (0, _LANES - _C))
    tc_out = _tc_part(feat, W, b2)
    sc_out = _sc_part(feat, W, b16)
    return jnp.concatenate([tc_out, sc_out[:, :_C]], axis=0)


# speedup vs baseline: 2.3807x; 1.0067x over previous
import jax
import jax.numpy as jnp
from jax import lax
from jax.experimental import pallas as pl
from jax.experimental.pallas import tpu as pltpu
from jax.experimental.pallas import tpu_sc as plsc

_B = 16384
_D = 2048
_C = 10
_B_SC = 2048
_B_TC = _B - _B_SC
_N_TILES = 32
_ROWS_PER_TILE = _B_SC // _N_TILES
_CHUNK = 16
_NCH = _ROWS_PER_TILE // _CHUNK
_LANES = 16


def _tc_kernel(f_ref, w_ref, b_ref, o_ref):
    acc = lax.dot_general(
        f_ref[...].astype(jnp.bfloat16), w_ref[...].astype(jnp.bfloat16),
        dimension_numbers=(((1,), (1,)), ((), ())),
        preferred_element_type=jnp.float32,
    )
    o_ref[...] = acc + b_ref[...]


def _tc_part(feat, W, b2):
    Bm = 1024
    return pl.pallas_call(
        _tc_kernel,
        grid=(_B_TC // Bm,),
        in_specs=[
            pl.BlockSpec((Bm, _D), lambda i: (i, 0)),
            pl.BlockSpec((_C, _D), lambda i: (0, 0)),
            pl.BlockSpec((1, _C), lambda i: (0, 0)),
        ],
        out_specs=pl.BlockSpec((Bm, _C), lambda i: (i, 0)),
        out_shape=jax.ShapeDtypeStruct((_B_TC, _C), jnp.float32),
        compiler_params=pltpu.CompilerParams(
            dimension_semantics=("arbitrary",),
        ),
    )(feat, W, b2)


def _sc_body(feat_hbm, w_hbm, b_hbm, out_hbm, wv, bv,
             fch0, fch1, ov0, ov1, fsem, osem):
    wid = lax.axis_index("s") * 2 + lax.axis_index("c")
    base = wid * _ROWS_PER_TILE

    pltpu.sync_copy(w_hbm, wv)
    pltpu.sync_copy(b_hbm, bv)
    bias = bv[...]
    lanes = lax.iota(jnp.int32, _LANES)
    zero = jnp.zeros((_LANES,), jnp.float32)
    fbufs = (fch0, fch1)
    obufs = (ov0, ov1)

    def in_copy(ch, buf, sem):
        row0 = base + ch * _CHUNK
        return pltpu.make_async_copy(
            feat_hbm.at[pl.ds(_B_TC + row0, _CHUNK), :], buf, sem)

    def out_copy(ch, buf, sem):
        row0 = base + ch * _CHUNK
        return pltpu.make_async_copy(
            buf, out_hbm.at[pl.ds(row0, _CHUNK), :], sem)

    def compute_chunk(fch, ov):
        for g in range(_CHUNK // 2):
            rows = (2 * g, 2 * g + 1)

            def d_body(d, accs, _rows=rows):
                sl = pl.ds(d * _LANES, _LANES)
                fs = [fch[r, sl] for r in _rows]
                new = list(accs)
                for c in range(_C):
                    wc = wv[c, sl]
                    for ri in range(2):
                        k = ri * _C + c
                        new[k] = new[k] + fs[ri] * wc
                return tuple(new)

            init = tuple(jnp.zeros((_LANES,), jnp.float32)
                         for _ in range(2 * _C))
            accs = lax.fori_loop(0, _D // _LANES, d_body, init, unroll=2)

            for ri in range(2):
                res = bias
                for c in range(_C):
                    s = jnp.sum(accs[ri * _C + c])
                    res = res + jnp.where(lanes == c,
                                          jnp.full((_LANES,), s), zero)
                ov[rows[ri]] = res

    in_copy(0, fch0, fsem.at[0]).start()
    nk = _NCH // 2

    def k_body(k, carry):
        ch0 = 2 * k
        ch1 = 2 * k + 1

        in_copy(ch1, fch1, fsem.at[1]).start()
        in_copy(ch0, fch0, fsem.at[0]).wait()

        @pl.when(k >= 1)
        def _drain0():
            out_copy(ch0 - 2, ov0, osem.at[0]).wait()

        compute_chunk(fch0, ov0)
        out_copy(ch0, ov0, osem.at[0]).start()

        @pl.when(k + 1 < nk)
        def _ahead():
            in_copy(ch0 + 2, fch0, fsem.at[0]).start()

        in_copy(ch1, fch1, fsem.at[1]).wait()

        @pl.when(k >= 1)
        def _drain1():
            out_copy(ch1 - 2, ov1, osem.at[1]).wait()

        compute_chunk(fch1, ov1)
        out_copy(ch1, ov1, osem.at[1]).start()
        return carry

    lax.fori_loop(0, nk, k_body, 0)

    out_copy(_NCH - 2, ov0, osem.at[0]).wait()
    out_copy(_NCH - 1, ov1, osem.at[1]).wait()


_sc_part = pl.kernel(
    _sc_body,
    out_type=jax.ShapeDtypeStruct((_B_SC, _LANES), jnp.float32),
    mesh=plsc.VectorSubcoreMesh(core_axis_name="c", subcore_axis_name="s"),
    scratch_types=[
        pltpu.VMEM((_C, _D), jnp.float32),
        pltpu.VMEM((_LANES,), jnp.float32),
        pltpu.VMEM((_CHUNK, _D), jnp.float32),
        pltpu.VMEM((_CHUNK, _D), jnp.float32),
        pltpu.VMEM((_CHUNK, _LANES), jnp.float32),
        pltpu.VMEM((_CHUNK, _LANES), jnp.float32),
        pltpu.SemaphoreType.DMA((2,)),
        pltpu.SemaphoreType.DMA((2,)),
    ],
    compiler_params=pltpu.CompilerParams(needs_layout_passes=False),
)


def kernel(feat, W, b):
    b2 = b.reshape(1, _C)
    b16 = jnp.pad(b, (0, _LANES - _C))
    tc_out = _tc_part(feat, W, b2)
    sc_out = _sc_part(feat, W, b16)
    return jnp.concatenate([tc_out, sc_out[:, :_C]], axis=0)


# speedup vs baseline: 2.4544x; 1.0310x over previous
import jax
import jax.numpy as jnp
from jax import lax
from jax.experimental import pallas as pl
from jax.experimental.pallas import tpu as pltpu
from jax.experimental.pallas import tpu_sc as plsc

_B = 16384
_D = 2048
_C = 10
_B_SC = 1024
_B_TC = _B - _B_SC
_N_TILES = 32
_ROWS_PER_TILE = _B_SC // _N_TILES
_CHUNK = 16
_NCH = _ROWS_PER_TILE // _CHUNK
_LANES = 16


def _tc_kernel(f_ref, w_ref, b_ref, o_ref):
    acc = lax.dot_general(
        f_ref[...].astype(jnp.bfloat16), w_ref[...].astype(jnp.bfloat16),
        dimension_numbers=(((1,), (1,)), ((), ())),
        preferred_element_type=jnp.float32,
    )
    o_ref[...] = acc + b_ref[...]


def _tc_part(feat, W, b2):
    Bm = 1024
    return pl.pallas_call(
        _tc_kernel,
        grid=(_B_TC // Bm,),
        in_specs=[
            pl.BlockSpec((Bm, _D), lambda i: (i, 0)),
            pl.BlockSpec((_C, _D), lambda i: (0, 0)),
            pl.BlockSpec((1, _C), lambda i: (0, 0)),
        ],
        out_specs=pl.BlockSpec((Bm, _C), lambda i: (i, 0)),
        out_shape=jax.ShapeDtypeStruct((_B_TC, _C), jnp.float32),
        compiler_params=pltpu.CompilerParams(
            dimension_semantics=("arbitrary",),
        ),
    )(feat, W, b2)


def _sc_body(feat_hbm, w_hbm, b_hbm, out_hbm, wv, bv,
             fch0, fch1, ov0, ov1, fsem, osem):
    wid = lax.axis_index("s") * 2 + lax.axis_index("c")
    base = wid * _ROWS_PER_TILE

    pltpu.sync_copy(w_hbm, wv)
    pltpu.sync_copy(b_hbm, bv)
    bias = bv[...]
    lanes = lax.iota(jnp.int32, _LANES)
    zero = jnp.zeros((_LANES,), jnp.float32)
    fbufs = (fch0, fch1)
    obufs = (ov0, ov1)

    def in_copy(ch, buf, sem):
        row0 = base + ch * _CHUNK
        return pltpu.make_async_copy(
            feat_hbm.at[pl.ds(_B_TC + row0, _CHUNK), :], buf, sem)

    def out_copy(ch, buf, sem):
        row0 = base + ch * _CHUNK
        return pltpu.make_async_copy(
            buf, out_hbm.at[pl.ds(row0, _CHUNK), :], sem)

    def compute_chunk(fch, ov):
        for g in range(_CHUNK // 2):
            rows = (2 * g, 2 * g + 1)

            def d_body(d, accs, _rows=rows):
                sl = pl.ds(d * _LANES, _LANES)
                fs = [fch[r, sl] for r in _rows]
                new = list(accs)
                for c in range(_C):
                    wc = wv[c, sl]
                    for ri in range(2):
                        k = ri * _C + c
                        new[k] = new[k] + fs[ri] * wc
                return tuple(new)

            init = tuple(jnp.zeros((_LANES,), jnp.float32)
                         for _ in range(2 * _C))
            accs = lax.fori_loop(0, _D // _LANES, d_body, init, unroll=2)

            for ri in range(2):
                res = bias
                for c in range(_C):
                    s = jnp.sum(accs[ri * _C + c])
                    res = res + jnp.where(lanes == c,
                                          jnp.full((_LANES,), s), zero)
                ov[rows[ri]] = res

    in_copy(0, fch0, fsem.at[0]).start()
    nk = _NCH // 2

    def k_body(k, carry):
        ch0 = 2 * k
        ch1 = 2 * k + 1

        in_copy(ch1, fch1, fsem.at[1]).start()
        in_copy(ch0, fch0, fsem.at[0]).wait()

        @pl.when(k >= 1)
        def _drain0():
            out_copy(ch0 - 2, ov0, osem.at[0]).wait()

        compute_chunk(fch0, ov0)
        out_copy(ch0, ov0, osem.at[0]).start()

        @pl.when(k + 1 < nk)
        def _ahead():
            in_copy(ch0 + 2, fch0, fsem.at[0]).start()

        in_copy(ch1, fch1, fsem.at[1]).wait()

        @pl.when(k >= 1)
        def _drain1():
            out_copy(ch1 - 2, ov1, osem.at[1]).wait()

        compute_chunk(fch1, ov1)
        out_copy(ch1, ov1, osem.at[1]).start()
        return carry

    lax.fori_loop(0, nk, k_body, 0)

    out_copy(_NCH - 2, ov0, osem.at[0]).wait()
    out_copy(_NCH - 1, ov1, osem.at[1]).wait()


_sc_part = pl.kernel(
    _sc_body,
    out_type=jax.ShapeDtypeStruct((_B_SC, _LANES), jnp.float32),
    mesh=plsc.VectorSubcoreMesh(core_axis_name="c", subcore_axis_name="s"),
    scratch_types=[
        pltpu.VMEM((_C, _D), jnp.float32),
        pltpu.VMEM((_LANES,), jnp.float32),
        pltpu.VMEM((_CHUNK, _D), jnp.float32),
        pltpu.VMEM((_CHUNK, _D), jnp.float32),
        pltpu.VMEM((_CHUNK, _LANES), jnp.float32),
        pltpu.VMEM((_CHUNK, _LANES), jnp.float32),
        pltpu.SemaphoreType.DMA((2,)),
        pltpu.SemaphoreType.DMA((2,)),
    ],
    compiler_params=pltpu.CompilerParams(needs_layout_passes=False),
)


def kernel(feat, W, b):
    b2 = b.reshape(1, _C)
    b16 = jnp.pad(b, (0, _LANES - _C))
    tc_out = _tc_part(feat, W, b2)
    sc_out = _sc_part(feat, W, b16)
    return jnp.concatenate([tc_out, sc_out[:, :_C]], axis=0)


# speedup vs baseline: 3.4975x; 1.4250x over previous
import jax
import jax.numpy as jnp
from jax import lax
from jax.experimental import pallas as pl
from jax.experimental.pallas import tpu as pltpu


def _linear_kernel(f_ref, w_ref, b_ref, o_ref):
    acc = lax.dot_general(
        f_ref[...].astype(jnp.bfloat16), w_ref[...].astype(jnp.bfloat16),
        dimension_numbers=(((1,), (1,)), ((), ())),
        preferred_element_type=jnp.float32,
    )
    o_ref[...] = acc + b_ref[...]


def kernel(feat, W, b):
    B, D = feat.shape
    C = W.shape[0]
    Bm = 1024
    return pl.pallas_call(
        _linear_kernel,
        grid=(B // Bm,),
        in_specs=[
            pl.BlockSpec((Bm, D), lambda i: (i, 0)),
            pl.BlockSpec((C, D), lambda i: (0, 0)),
            pl.BlockSpec((1, C), lambda i: (0, 0)),
        ],
        out_specs=pl.BlockSpec((Bm, C), lambda i: (i, 0)),
        out_shape=jax.ShapeDtypeStruct((B, C), jnp.float32),
        compiler_params=pltpu.CompilerParams(
            dimension_semantics=("arbitrary",),
        ),
    )(feat, W, b.reshape(1, C))
